# baseline (speedup 1.0000x reference)
import functools

import jax
import jax.numpy as jnp
from jax import lax
from jax.experimental import pallas as pl
from jax.experimental.pallas import tpu as pltpu
from jax.experimental.pallas import tpu_sc as plsc

_G = 128


def _build(N, V, D, NC, NS):
    NW = NC * NS
    n_per_w = N // NW
    G = _G
    n_groups = n_per_w // G
    NBUF = 2

    mesh = plsc.VectorSubcoreMesh(core_axis_name="c", subcore_axis_name="s")

    @functools.partial(
        pl.kernel,
        out_type=jax.ShapeDtypeStruct((N, D), jnp.float32),
        mesh=mesh,
        scratch_types=[
            pltpu.VMEM((n_groups, G), jnp.int32),
            pltpu.VMEM((NBUF, G, D), jnp.float32),
            [pltpu.SemaphoreType.DMA] * NBUF,
        ],
    )
    def k(idx_hbm, table_hbm, out_hbm, idx_v, rows_v, gsems):
        c = lax.axis_index("c")
        s = lax.axis_index("s")
        wid = s * NC + c
        base = wid * n_per_w

        pltpu.sync_copy(idx_hbm.at[wid], idx_v)

        for b in range(NBUF):
            pltpu.async_copy(table_hbm.at[idx_v.at[b]], rows_v.at[b], gsems[b])

        @pl.loop(0, n_groups, step=NBUF)
        def _(j):
            for b in range(NBUF):
                t = j + b
                pltpu.make_async_copy(
                    table_hbm.at[idx_v.at[t]], rows_v.at[b], gsems[b]
                ).wait()
                pltpu.sync_copy(rows_v.at[b], out_hbm.at[pl.ds(base + t * G, G)])

                @pl.when(t + NBUF < n_groups)
                def _():
                    pltpu.async_copy(
                        table_hbm.at[idx_v.at[t + NBUF]], rows_v.at[b], gsems[b]
                    )

    return k


def kernel(x, table):
    B, H = x.shape
    V, D = table.shape
    N = B * H
    info = plsc.get_sparse_core_info()
    NC, NS = info.num_cores, info.num_subcores
    NW = NC * NS
    n_per_w = N // NW
    idx = x.T.reshape(NW, n_per_w // _G, _G)
    out = _build(N, V, D, NC, NS)(idx, table)
    return out.reshape(H, B, D).transpose(1, 0, 2)


# speedup vs baseline: 1.0041x; 1.0041x over previous
import functools

import jax
import jax.numpy as jnp
from jax import lax
from jax.experimental import pallas as pl
from jax.experimental.pallas import tpu as pltpu
from jax.experimental.pallas import tpu_sc as plsc

_G = 128


def _build(N, V, D, NC, NS):
    NW = NC * NS
    n_per_w = N // NW
    G = _G
    n_groups = n_per_w // G
    NBUF = 5

    mesh = plsc.VectorSubcoreMesh(core_axis_name="c", subcore_axis_name="s")

    @functools.partial(
        pl.kernel,
        out_type=jax.ShapeDtypeStruct((N, D), jnp.float32),
        mesh=mesh,
        scratch_types=[
            pltpu.VMEM((n_groups, G), jnp.int32),
            pltpu.VMEM((NBUF, G, D), jnp.float32),
            [pltpu.SemaphoreType.DMA] * NBUF,
        ],
    )
    def k(idx_hbm, table_hbm, out_hbm, idx_v, rows_v, gsems):
        c = lax.axis_index("c")
        s = lax.axis_index("s")
        wid = s * NC + c
        base = wid * n_per_w

        pltpu.sync_copy(idx_hbm.at[wid], idx_v)

        for b in range(NBUF):
            pltpu.async_copy(table_hbm.at[idx_v.at[b]], rows_v.at[b], gsems[b])

        @pl.loop(0, n_groups, step=NBUF)
        def _(j):
            for b in range(NBUF):
                t = j + b
                pltpu.make_async_copy(
                    table_hbm.at[idx_v.at[t]], rows_v.at[b], gsems[b]
                ).wait()
                pltpu.sync_copy(rows_v.at[b], out_hbm.at[pl.ds(base + t * G, G)])

                @pl.when(t + NBUF < n_groups)
                def _():
                    pltpu.async_copy(
                        table_hbm.at[idx_v.at[t + NBUF]], rows_v.at[b], gsems[b]
                    )

    return k


def kernel(x, table):
    B, H = x.shape
    V, D = table.shape
    N = B * H
    info = plsc.get_sparse_core_info()
    NC, NS = info.num_cores, info.num_subcores
    NW = NC * NS
    n_per_w = N // NW
    idx = x.T.reshape(NW, n_per_w // _G, _G)
    out = _build(N, V, D, NC, NS)(idx, table)
    return out.reshape(H, B, D).transpose(1, 0, 2)
